# baseline (speedup 1.0000x reference)
import functools

import jax
import jax.numpy as jnp
from jax import lax
from jax.experimental import pallas as pl
from jax.experimental.pallas import tpu as pltpu
from jax.experimental.pallas import tpu_sc as plsc

VOCAB = 1000000
EMBED = 64
NTOK = 200
BATCH = 4096

NUM_CORES = 2
NUM_SUBCORES = 16
NW = NUM_CORES * NUM_SUBCORES
ROWS_PER_W = BATCH // NW

@functools.lru_cache(maxsize=1)
def _build_kernel():
    mesh = plsc.VectorSubcoreMesh(core_axis_name="c", subcore_axis_name="s")

    @functools.partial(
        pl.kernel,
        mesh=mesh,
        out_type=jax.ShapeDtypeStruct((BATCH, NTOK, EMBED), jnp.float32),
        compiler_params=pltpu.CompilerParams(use_tc_tiling_on_sc=False),
        scratch_types=[
            pltpu.VMEM((NTOK,), jnp.int32),
            pltpu.VMEM((NTOK, EMBED), jnp.float32),
            pltpu.VMEM((NTOK, EMBED), jnp.float32),
            pltpu.SemaphoreType.DMA,
        ],
    )
    def _embed_kernel(tokens_hbm, table_hbm, pos_hbm, out_hbm,
                      idx_v, rows_v, pos_v, sem):
        wid = lax.axis_index("s") * NUM_CORES + lax.axis_index("c")
        base = wid * ROWS_PER_W
        pltpu.sync_copy(pos_hbm, pos_v)

        def body(r, carry):
            row = base + r
            pltpu.sync_copy(tokens_hbm.at[row], idx_v)
            cp1 = pltpu.async_copy(
                table_hbm.at[idx_v.at[pl.ds(0, 128)]],
                rows_v.at[pl.ds(0, 128)], sem)
            cp2 = pltpu.async_copy(
                table_hbm.at[idx_v.at[pl.ds(128, NTOK - 128)]],
                rows_v.at[pl.ds(128, NTOK - 128)], sem)
            cp1.wait()
            cp2.wait()

            def add_row(t, c):
                for j in range(EMBED // 16):
                    sl = pl.ds(j * 16, 16)
                    rows_v[t, sl] = rows_v[t, sl] + pos_v[t, sl]
                return c

            lax.fori_loop(0, NTOK, add_row, 0)
            pltpu.sync_copy(rows_v, out_hbm.at[row])
            return carry

        lax.fori_loop(0, ROWS_PER_W, body, 0)

    return _embed_kernel


def kernel(tokens, token_embedding, position_embedding):
    return _build_kernel()(tokens, token_embedding, position_embedding)


# speedup vs baseline: 1.1915x; 1.1915x over previous
import functools

import jax
import jax.numpy as jnp
from jax import lax
from jax.experimental import pallas as pl
from jax.experimental.pallas import tpu as pltpu
from jax.experimental.pallas import tpu_sc as plsc

VOCAB = 1000000
EMBED = 64
NTOK = 200
BATCH = 4096

NUM_CORES = 2
NUM_SUBCORES = 16
NW = NUM_CORES * NUM_SUBCORES
ROWS_PER_W = BATCH // NW

_GATHER_SPLITS = ((0, 128), (128, NTOK - 128))


@functools.lru_cache(maxsize=1)
def _build_kernel():
    mesh = plsc.VectorSubcoreMesh(core_axis_name="c", subcore_axis_name="s")

    @functools.partial(
        pl.kernel,
        mesh=mesh,
        out_type=jax.ShapeDtypeStruct((NW, ROWS_PER_W, NTOK, EMBED),
                                      jnp.float32),
        compiler_params=pltpu.CompilerParams(use_tc_tiling_on_sc=False),
        scratch_types=[
            pltpu.VMEM((ROWS_PER_W, NTOK), jnp.int32),
            pltpu.VMEM((NTOK, EMBED), jnp.float32),
            pltpu.VMEM((NTOK, EMBED), jnp.float32),
            pltpu.VMEM((NTOK, EMBED), jnp.float32),
            pltpu.VMEM((NTOK, EMBED), jnp.float32),
            pltpu.VMEM((NTOK, EMBED), jnp.float32),
            pltpu.SemaphoreType.DMA,
            pltpu.SemaphoreType.DMA,
            pltpu.SemaphoreType.DMA,
            pltpu.SemaphoreType.DMA,
        ],
    )
    def _embed_kernel(tokens_hbm, table_hbm, pos_hbm, out_hbm,
                      idx_v, pos_v, gbuf0, gbuf1, sbuf0, sbuf1,
                      gsem0, gsem1, ssem0, ssem1):
        gbufs = (gbuf0, gbuf1)
        sbufs = (sbuf0, sbuf1)
        gsems = (gsem0, gsem1)
        ssems = (ssem0, ssem1)
        wid = lax.axis_index("s") * NUM_CORES + lax.axis_index("c")

        pltpu.sync_copy(pos_hbm, pos_v)
        pltpu.sync_copy(tokens_hbm.at[wid], idx_v)

        def issue_gather(c, b):
            for off, sz in _GATHER_SPLITS:
                pltpu.async_copy(
                    table_hbm.at[idx_v.at[c].at[pl.ds(off, sz)]],
                    gbufs[b].at[pl.ds(off, sz)], gsems[b])

        def wait_gather(b):
            pltpu.make_async_copy(
                table_hbm.at[pl.ds(0, NTOK)], gbufs[b], gsems[b]).wait()

        def issue_scatter(c, b):
            pltpu.async_copy(sbufs[b], out_hbm.at[wid, c], ssems[b])

        def wait_scatter(c, b):
            pltpu.make_async_copy(
                sbufs[b], out_hbm.at[wid, c], ssems[b]).wait()

        issue_gather(0, 0)
        issue_gather(1, 1)

        def body(i, carry):
            for b in range(2):
                c = i * 2 + b
                wait_gather(b)

                @pl.when(i > 0)
                def _():
                    wait_scatter(c - 2, b)

                def add_row(t, acc):
                    for j in range(EMBED // 16):
                        sl = pl.ds(j * 16, 16)
                        sbufs[b][t, sl] = gbufs[b][t, sl] + pos_v[t, sl]
                    return acc

                lax.fori_loop(0, NTOK, add_row, 0)

                @pl.when(i < (ROWS_PER_W // 2) - 1)
                def _():
                    issue_gather(c + 2, b)

                issue_scatter(c, b)
            return carry

        lax.fori_loop(0, ROWS_PER_W // 2, body, 0)
        wait_scatter(ROWS_PER_W - 2, 0)
        wait_scatter(ROWS_PER_W - 1, 1)

    return _embed_kernel


def kernel(tokens, token_embedding, position_embedding):
    tok = tokens.reshape(NW, ROWS_PER_W, NTOK)
    out = _build_kernel()(tok, token_embedding, position_embedding)
    return out.reshape(BATCH, NTOK, EMBED)
